# Initial kernel scaffold; baseline (speedup 1.0000x reference)
import jax
import jax.numpy as jnp
from jax.experimental import pallas as pl


def kernel(x, gate_w, Wg, Wu, Wd):
    raise NotImplementedError("write your pallas kernel here")


# speedup vs baseline: 1.2193x; 1.2193x over previous
import functools

import jax
import jax.numpy as jnp
from jax.experimental import pallas as pl
from jax.experimental.pallas import tpu as pltpu

DIM = 2048
HID = 1024
NE = 8
TB = 512

INTERPRET = False


def _router_kernel(x_ref, gwt_ref, w_ref, xb_ref):
    x = x_ref[...]
    xb = x.astype(jnp.bfloat16)
    xb_ref[...] = xb
    logits = jnp.dot(xb, gwt_ref[...].astype(jnp.bfloat16),
                     preferred_element_type=jnp.float32)
    lane = jax.lax.broadcasted_iota(jnp.int32, (TB, 128), 1)
    neg = jnp.float32(-1e30)
    logits = jnp.where(lane < NE, logits, neg)
    m0 = jnp.max(logits, axis=1, keepdims=True)
    i0 = jnp.min(jnp.where(logits == m0, lane, 127), axis=1, keepdims=True)
    logits1 = jnp.where(lane == i0, neg, logits)
    m1 = jnp.max(logits1, axis=1, keepdims=True)
    i1 = jnp.min(jnp.where(logits1 == m1, lane, 127), axis=1, keepdims=True)
    s0 = jax.nn.sigmoid(m0)
    s1 = jax.nn.sigmoid(m1)
    denom = s0 + s1 + jnp.float32(1e-9)
    w0 = s0 / denom
    w1 = s1 / denom
    w = jnp.where(lane == i0, w0, 0.0) + jnp.where(lane == i1, w1, 0.0)
    w_ref[...] = w.astype(jnp.float32)


def _dense_mlp_kernel(xb_ref, w_ref, wg_ref, wu_ref, wd_ref, out_ref, acc_ref):
    e = pl.program_id(1)

    @pl.when(e == 0)
    def _():
        acc_ref[...] = jnp.zeros_like(acc_ref)

    xb = xb_ref[...]
    dn = (((1,), (1,)), ((), ()))
    g = jax.lax.dot_general(xb, wg_ref[0], dn, preferred_element_type=jnp.float32)
    u = jax.lax.dot_general(xb, wu_ref[0], dn, preferred_element_type=jnp.float32)
    h = (g * jax.nn.sigmoid(g)) * u
    y = jax.lax.dot_general(h.astype(jnp.bfloat16), wd_ref[0], dn,
                            preferred_element_type=jnp.float32)
    lane = jax.lax.broadcasted_iota(jnp.int32, (TB, 128), 1)
    we = jnp.sum(jnp.where(lane == e, w_ref[...], 0.0), axis=1, keepdims=True)
    acc_ref[...] += y * we

    @pl.when(e == NE - 1)
    def _():
        out_ref[...] = acc_ref[...]


def kernel(x, gate_w, Wg, Wu, Wd):
    bsz, seqlen, dim = x.shape
    T = bsz * seqlen
    nb = T // TB
    flat = x.reshape(T, dim)
    gwt = jnp.zeros((DIM, 128), jnp.float32).at[:, :NE].set(gate_w.T)

    w, xb = pl.pallas_call(
        _router_kernel,
        grid=(nb,),
        in_specs=[
            pl.BlockSpec((TB, DIM), lambda i: (i, 0)),
            pl.BlockSpec((DIM, 128), lambda i: (0, 0)),
        ],
        out_specs=[
            pl.BlockSpec((TB, 128), lambda i: (i, 0)),
            pl.BlockSpec((TB, DIM), lambda i: (i, 0)),
        ],
        out_shape=[
            jax.ShapeDtypeStruct((T, 128), jnp.float32),
            jax.ShapeDtypeStruct((T, DIM), jnp.bfloat16),
        ],
        interpret=INTERPRET,
    )(flat, gwt)

    Wg_b = Wg.astype(jnp.bfloat16)
    Wu_b = Wu.astype(jnp.bfloat16)
    Wd_b = Wd.astype(jnp.bfloat16)

    out = pl.pallas_call(
        _dense_mlp_kernel,
        grid=(nb, NE),
        in_specs=[
            pl.BlockSpec((TB, DIM), lambda i, e: (i, 0)),
            pl.BlockSpec((TB, 128), lambda i, e: (i, 0)),
            pl.BlockSpec((1, HID, DIM), lambda i, e: (e, 0, 0)),
            pl.BlockSpec((1, HID, DIM), lambda i, e: (e, 0, 0)),
            pl.BlockSpec((1, DIM, HID), lambda i, e: (e, 0, 0)),
        ],
        out_specs=pl.BlockSpec((TB, DIM), lambda i, e: (i, 0)),
        out_shape=jax.ShapeDtypeStruct((T, DIM), jnp.float32),
        scratch_shapes=[pltpu.VMEM((TB, DIM), jnp.float32)],
        interpret=INTERPRET,
    )(xb, w, Wg_b, Wu_b, Wd_b)

    return out.reshape(bsz, seqlen, dim)
